# baseline (speedup 1.0000x reference)
import functools

import jax
import jax.numpy as jnp
from jax import lax
from jax.experimental import pallas as pl
from jax.experimental.pallas import tpu as pltpu
from jax.experimental.pallas import tpu_sc as plsc

_E = 8
_K = 2
_N = 2048
_D = 1024
_H = 512
_TBG = 256
_NP = 4096 + _E * _TBG
_NB = _NP // _TBG
_NBP = 64
_CH = 512


def _silu(v):
    return v / (1.0 + jnp.exp(-v))


def _as_bf16(v_i32):
    return lax.bitcast_convert_type(v_i32, jnp.bfloat16).reshape(
        v_i32.shape[0], v_i32.shape[1] * 2)


def _dot_t(a, b):
    return lax.dot_general(a, b, (((1,), (1,)), ((), ())),
                           preferred_element_type=jnp.float32)


def _routing_body(lg_ref, bias_ref, pos_ref, w_ref, meta_ref):
    lg = lg_ref[...]
    lb = lg + bias_ref[...]
    ii = lax.broadcasted_iota(jnp.int32, lb.shape, 1)
    m1 = jnp.max(lb, axis=1, keepdims=True)
    i1 = jnp.min(jnp.where(lb == m1, ii, _E), axis=1, keepdims=True)
    lb2 = jnp.where(ii == i1, -jnp.inf, lb)
    m2 = jnp.max(lb2, axis=1, keepdims=True)
    i2 = jnp.min(jnp.where(lb2 == m2, ii, _E), axis=1, keepdims=True)
    ex = jnp.exp(lg - jnp.max(lg, axis=1, keepdims=True))
    sc = ex / jnp.sum(ex, axis=1, keepdims=True)
    s1 = jnp.sum(jnp.where(ii == i1, sc, 0.0), axis=1, keepdims=True)
    s2 = jnp.sum(jnp.where(ii == i2, sc, 0.0), axis=1, keepdims=True)
    tot = s1 + s2
    w_ref[pl.ds(0, _N), :] = s1 / tot
    w_ref[pl.ds(_N, _N), :] = s2 / tot

    tri = (lax.broadcasted_iota(jnp.int32, (_CH, _CH), 0)
           > lax.broadcasted_iota(jnp.int32, (_CH, _CH), 1)).astype(jnp.bfloat16)
    lane = lax.broadcasted_iota(jnp.int32, (_CH, _E), 1)
    ohs, ranks = [], []
    run = jnp.zeros((1, _E), jnp.float32)
    ng = (_K * _N) // _CH
    for g in range(ng):
        src = i1 if g < ng // 2 else i2
        r0 = (g % (ng // 2)) * _CH
        e_blk = lax.slice(src, (r0, 0), (r0 + _CH, 1))
        oh = (lane == e_blk).astype(jnp.float32)
        rank = lax.dot_general(tri, oh.astype(jnp.bfloat16),
                               (((1,), (0,)), ((), ())),
                               preferred_element_type=jnp.float32) + run
        ohs.append(oh)
        ranks.append(rank)
        run = run + jnp.sum(oh, axis=0, keepdims=True)

    cnt = run
    nblk = jnp.floor((cnt + (_TBG - 1)) * (1.0 / _TBG))
    mlt = (lax.broadcasted_iota(jnp.int32, (_E, _E), 0)
           < lax.broadcasted_iota(jnp.int32, (_E, _E), 1)).astype(jnp.bfloat16)
    cumblk = lax.dot_general(nblk.astype(jnp.bfloat16), mlt,
                             (((1,), (0,)), ((), ())),
                             preferred_element_type=jnp.float32)
    offpad = cumblk * float(_TBG)

    for g in range(ng):
        posf = jnp.sum(ohs[g] * (ranks[g] + offpad), axis=1, keepdims=True)
        pos_ref[pl.ds(g * _CH, _CH), :] = posf.astype(jnp.int32)

    bi = lax.broadcasted_iota(jnp.int32, (_NBP, _E), 0).astype(jnp.float32)
    ind = (bi >= cumblk).astype(jnp.float32)
    be = jnp.sum(ind, axis=1, keepdims=True) - 1.0
    totblk = jnp.sum(nblk)
    act = (bi[:, :1] < totblk).astype(jnp.float32)
    meta_ref[pl.ds(0, _NBP), :] = be.astype(jnp.int32)
    meta_ref[pl.ds(_NBP, _NBP), :] = act.astype(jnp.int32)


def _routing(logits, router_bias):
    return pl.pallas_call(
        _routing_body,
        in_specs=[
            pl.BlockSpec((_N, _E), lambda: (0, 0)),
            pl.BlockSpec((1, _E), lambda: (0, 0)),
        ],
        out_specs=[
            pl.BlockSpec((_K * _N, 1), lambda: (0, 0)),
            pl.BlockSpec((_K * _N, 1), lambda: (0, 0)),
            pl.BlockSpec((2 * _NBP, 1), lambda: (0, 0)),
        ],
        out_shape=[
            jax.ShapeDtypeStruct((_K * _N, 1), jnp.int32),
            jax.ShapeDtypeStruct((_K * _N, 1), jnp.float32),
            jax.ShapeDtypeStruct((2 * _NBP, 1), jnp.int32),
        ],
    )(logits, router_bias.reshape(1, _E))


def _dispatch_body(pos_hbm, wv_hbm, tok_hbm, st_hbm, ws_hbm,
                   idx_v, w_v, t_v, semw, semt):
    wid = lax.axis_index("s") * 2 + lax.axis_index("c")
    npa = _K * _N // 32
    a0 = wid * npa
    pltpu.sync_copy(pos_hbm.at[pl.ds(a0, npa)], idx_v)
    pltpu.sync_copy(wv_hbm.at[pl.ds(a0, npa)], w_v)
    pltpu.sync_copy(tok_hbm.at[pl.ds(a0, npa)], t_v)
    cw = pltpu.async_copy(w_v, ws_hbm.at[idx_v], semw)
    ct = pltpu.async_copy(t_v, st_hbm.at[idx_v], semt)
    cw.wait()
    ct.wait()


def _dispatch_sc(pos_flat, w_flat, tok_flat):
    npa = _K * _N // 32
    mesh = plsc.VectorSubcoreMesh(core_axis_name="c", subcore_axis_name="s")
    f = functools.partial(
        pl.kernel,
        out_type=[
            jax.ShapeDtypeStruct((_NP,), jnp.int32),
            jax.ShapeDtypeStruct((_NP,), jnp.float32),
        ],
        mesh=mesh,
        scratch_types=[
            pltpu.VMEM((npa,), jnp.int32),
            pltpu.VMEM((npa,), jnp.float32),
            pltpu.VMEM((npa,), jnp.int32),
            pltpu.SemaphoreType.DMA,
            pltpu.SemaphoreType.DMA,
        ],
        compiler_params=pltpu.CompilerParams(use_tc_tiling_on_sc=True),
    )(_dispatch_body)
    return f(pos_flat, w_flat, tok_flat)


def _ffn_triple(x_blk, wu, wg, wd):
    up = _dot_t(x_blk, wu)
    gate = _dot_t(x_blk, wg)
    hid = _silu(up) * gate
    return lax.dot_general(hid.astype(jnp.bfloat16), wd,
                           (((1,), (0,)), ((), ())),
                           preferred_element_type=jnp.float32)


def _grouped_body(sp_ref, xbf_ref, st_ref, ws_ref, wu_ref, wg_ref, wd_ref,
                  su_ref, sg_ref, sd_ref, eo_ref, sh_ref):
    b = pl.program_id(0)
    act = sp_ref[_NBP + b]

    @pl.when(b < _NB)
    def _():
        @pl.when(act == 1)
        def _():
            st_blk = st_ref[...]
            toks = lax.broadcasted_iota(jnp.int32, (_TBG, _N), 1)
            p = (toks == st_blk).astype(jnp.bfloat16)
            xg = lax.dot_general(p, xbf_ref[...],
                                 (((1,), (0,)), ((), ())),
                                 preferred_element_type=jnp.float32)
            x_blk = xg.astype(jnp.bfloat16)
            eo = _ffn_triple(x_blk, wu_ref[0], wg_ref[0], wd_ref[0])
            eo_ref[...] = eo * ws_ref[...]

        @pl.when(act == 0)
        def _():
            eo_ref[...] = jnp.zeros_like(eo_ref)

    @pl.when(b >= _NB)
    def _():
        tb = b - _NB
        x_blk = xbf_ref[pl.ds(tb * _TBG, _TBG), :]
        acc = None
        for s in range(su_ref.shape[0]):
            eo = _ffn_triple(x_blk, su_ref[s], sg_ref[s], sd_ref[s])
            acc = eo if acc is None else acc + eo
        sh_ref[...] = acc


def _grouped(meta, xbf, st_col, ws_col, up_r, gate_r, down_r,
             s_up, s_gate, s_down):
    nbt = _NB + _N // _TBG
    grid_spec = pltpu.PrefetchScalarGridSpec(
        num_scalar_prefetch=1,
        grid=(nbt,),
        in_specs=[
            pl.BlockSpec((_N, _D), lambda b, sp: (0, 0)),
            pl.BlockSpec((_TBG, 1),
                         lambda b, sp: (jnp.minimum(b, _NB - 1), 0)),
            pl.BlockSpec((_TBG, 1),
                         lambda b, sp: (jnp.minimum(b, _NB - 1), 0)),
            pl.BlockSpec((1, _H, _D), lambda b, sp: (sp[b], 0, 0)),
            pl.BlockSpec((1, _H, _D), lambda b, sp: (sp[b], 0, 0)),
            pl.BlockSpec((1, _H, _D), lambda b, sp: (sp[b], 0, 0)),
            pl.BlockSpec((2, _H, _D), lambda b, sp: (0, 0, 0)),
            pl.BlockSpec((2, _H, _D), lambda b, sp: (0, 0, 0)),
            pl.BlockSpec((2, _H, _D), lambda b, sp: (0, 0, 0)),
        ],
        out_specs=[
            pl.BlockSpec((_TBG, _D),
                         lambda b, sp: (jnp.minimum(b, _NB - 1), 0)),
            pl.BlockSpec((_TBG, _D),
                         lambda b, sp: (jnp.maximum(b - _NB, 0), 0)),
        ],
    )
    return pl.pallas_call(
        _grouped_body,
        grid_spec=grid_spec,
        out_shape=[
            jax.ShapeDtypeStruct((_NP, _D), jnp.float32),
            jax.ShapeDtypeStruct((_N, _D), jnp.float32),
        ],
        compiler_params=pltpu.CompilerParams(
            dimension_semantics=("arbitrary",)),
    )(meta.reshape(-1), xbf, st_col, ws_col, up_r, gate_r, down_r,
      s_up, s_gate, s_down)


_CSZ = 8


def _combine_body(eo_hbm, sh_hbm, pos_hbm, out_hbm,
                  idx0a, idx1a, g0a, g1a, sha,
                  idx0b, idx1b, g0b, g1b, shb,
                  o_v, sem0a, sem1a, semsa, sem0b, sem1b, semsb):
    wid = lax.axis_index("s") * 2 + lax.axis_index("c")
    tpw = _N // 32
    nch = tpw // _CSZ
    bufs = [(idx0a, idx1a, g0a, g1a, sha, sem0a, sem1a, semsa),
            (idx0b, idx1b, g0b, g1b, shb, sem0b, sem1b, semsb)]

    def start(ch):
        i0, i1, g0, g1, sh, s0, s1, ss = bufs[ch % 2]
        t0 = wid * tpw + ch * _CSZ
        pltpu.sync_copy(pos_hbm.at[pl.ds(t0, _CSZ)], i0)
        pltpu.sync_copy(pos_hbm.at[pl.ds(_N + t0, _CSZ)], i1)
        c0 = pltpu.async_copy(eo_hbm.at[i0], g0, s0)
        c1 = pltpu.async_copy(eo_hbm.at[i1], g1, s1)
        cs = pltpu.async_copy(sh_hbm.at[pl.ds(t0, _CSZ)], sh, ss)
        return c0, c1, cs

    pend = start(0)
    for ch in range(nch):
        nxt = start(ch + 1) if ch + 1 < nch else None
        for c in pend:
            c.wait()
        _, _, g0, g1, sh, _, _, _ = bufs[ch % 2]

        def body(c, carry):
            for r in range(_CSZ):
                sl = pl.ds(c * 16, 16)
                o_v[r, sl] = g0[r, sl] + g1[r, sl] + sh[r, sl]
            return carry

        lax.fori_loop(0, _D // 16, body, 0)
        t0 = wid * tpw + ch * _CSZ
        pltpu.sync_copy(o_v, out_hbm.at[pl.ds(t0, _CSZ)])
        pend = nxt


def _combine_sc(eo, shared_out, pos_flat):
    mesh = plsc.VectorSubcoreMesh(core_axis_name="c", subcore_axis_name="s")
    buf = lambda: pltpu.VMEM((_CSZ, _D), jnp.float32)
    idx = lambda: pltpu.VMEM((_CSZ,), jnp.int32)
    f = functools.partial(
        pl.kernel,
        out_type=jax.ShapeDtypeStruct((_N, _D), jnp.float32),
        mesh=mesh,
        scratch_types=[
            idx(), idx(), buf(), buf(), buf(),
            idx(), idx(), buf(), buf(), buf(),
            buf(),
            pltpu.SemaphoreType.DMA, pltpu.SemaphoreType.DMA,
            pltpu.SemaphoreType.DMA, pltpu.SemaphoreType.DMA,
            pltpu.SemaphoreType.DMA, pltpu.SemaphoreType.DMA,
        ],
        compiler_params=pltpu.CompilerParams(use_tc_tiling_on_sc=True),
    )(_combine_body)
    return f(eo, shared_out, pos_flat)


@jax.jit
def kernel(x, Wr, router_bias, W_up, W_gate, W_down, Ws_up, Ws_gate, Ws_down):
    Bn, Tn, Dn = x.shape
    bf = jnp.bfloat16
    flat = x.reshape(_N, _D)
    logits = flat @ Wr.T
    xbf = flat.astype(bf)

    up_r = W_up.astype(bf)
    gate_r = W_gate.astype(bf)
    down_r = W_down.astype(bf).transpose(0, 2, 1)
    s_up = Ws_up[0].astype(bf).reshape(2, _H, _D)
    s_gate = Ws_gate[0].astype(bf).reshape(2, _H, _D)
    s_down = Ws_down[0].astype(bf).T.reshape(2, _H, _D)

    pos, wv, meta = _routing(logits, router_bias)
    pos_flat = pos.reshape(-1)
    w_flat = wv.reshape(-1)

    tok_flat = jnp.tile(jnp.arange(_N, dtype=jnp.int32), _K)
    st, ws = _dispatch_sc(pos_flat, w_flat, tok_flat)
    eo, sh = _grouped(meta, xbf, st.reshape(_NP, 1), ws.reshape(_NP, 1),
                      up_r, gate_r, down_r, s_up, s_gate, s_down)
    out = _combine_sc(eo, sh, pos_flat)
    return out.reshape(Bn, Tn, Dn)


# speedup vs baseline: 1.1050x; 1.1050x over previous
import functools

import jax
import jax.numpy as jnp
from jax import lax
from jax.experimental import pallas as pl
from jax.experimental.pallas import tpu as pltpu
from jax.experimental.pallas import tpu_sc as plsc

_E = 8
_K = 2
_N = 2048
_D = 1024
_H = 512
_TBG = 256
_NP = 4096 + _E * _TBG
_NB = _NP // _TBG
_NBP = 64
_CH = 512
_RW = 128


def _silu(v):
    return v / (1.0 + jnp.exp(-v))


def _as_bf16(v_i32):
    return lax.bitcast_convert_type(v_i32, jnp.bfloat16).reshape(
        v_i32.shape[0], v_i32.shape[1] * 2)


def _dot_t(a, b):
    return lax.dot_general(a, b, (((1,), (1,)), ((), ())),
                           preferred_element_type=jnp.float32)


def _routing_body(lg_ref, bias_ref, pos_ref, rec_ref, meta_ref):
    lg = lg_ref[...]
    lb = lg + bias_ref[...]
    ii = lax.broadcasted_iota(jnp.int32, lb.shape, 1)
    m1 = jnp.max(lb, axis=1, keepdims=True)
    i1 = jnp.min(jnp.where(lb == m1, ii, _E), axis=1, keepdims=True)
    lb2 = jnp.where(ii == i1, -jnp.inf, lb)
    m2 = jnp.max(lb2, axis=1, keepdims=True)
    i2 = jnp.min(jnp.where(lb2 == m2, ii, _E), axis=1, keepdims=True)
    ex = jnp.exp(lg - jnp.max(lg, axis=1, keepdims=True))
    sc = ex / jnp.sum(ex, axis=1, keepdims=True)
    s1 = jnp.sum(jnp.where(ii == i1, sc, 0.0), axis=1, keepdims=True)
    s2 = jnp.sum(jnp.where(ii == i2, sc, 0.0), axis=1, keepdims=True)
    tot = s1 + s2
    lane16 = lax.broadcasted_iota(jnp.int32, (_N, _RW), 1)
    ti = lax.broadcasted_iota(jnp.int32, (_N, _RW), 0).astype(jnp.float32)
    rec_ref[pl.ds(0, _N), :] = jnp.where(
        lane16 == 0, ti, jnp.where(lane16 == 1, s1 / tot, 0.0))
    rec_ref[pl.ds(_N, _N), :] = jnp.where(
        lane16 == 0, ti, jnp.where(lane16 == 1, s2 / tot, 0.0))

    tri = (lax.broadcasted_iota(jnp.int32, (_CH, _CH), 0)
           > lax.broadcasted_iota(jnp.int32, (_CH, _CH), 1)).astype(jnp.bfloat16)
    lane = lax.broadcasted_iota(jnp.int32, (_CH, _E), 1)
    ohs, ranks = [], []
    run = jnp.zeros((1, _E), jnp.float32)
    ng = (_K * _N) // _CH
    for g in range(ng):
        src = i1 if g < ng // 2 else i2
        r0 = (g % (ng // 2)) * _CH
        e_blk = lax.slice(src, (r0, 0), (r0 + _CH, 1))
        oh = (lane == e_blk).astype(jnp.float32)
        rank = lax.dot_general(tri, oh.astype(jnp.bfloat16),
                               (((1,), (0,)), ((), ())),
                               preferred_element_type=jnp.float32) + run
        ohs.append(oh)
        ranks.append(rank)
        run = run + jnp.sum(oh, axis=0, keepdims=True)

    cnt = run
    nblk = jnp.floor((cnt + (_TBG - 1)) * (1.0 / _TBG))
    mlt = (lax.broadcasted_iota(jnp.int32, (_E, _E), 0)
           < lax.broadcasted_iota(jnp.int32, (_E, _E), 1)).astype(jnp.bfloat16)
    cumblk = lax.dot_general(nblk.astype(jnp.bfloat16), mlt,
                             (((1,), (0,)), ((), ())),
                             preferred_element_type=jnp.float32)
    offpad = cumblk * float(_TBG)

    for g in range(ng):
        posf = jnp.sum(ohs[g] * (ranks[g] + offpad), axis=1, keepdims=True)
        pos_ref[pl.ds(g * _CH, _CH), :] = posf.astype(jnp.int32)

    bi = lax.broadcasted_iota(jnp.int32, (_NBP, _E), 0).astype(jnp.float32)
    ind = (bi >= cumblk).astype(jnp.float32)
    be = jnp.sum(ind, axis=1, keepdims=True) - 1.0
    totblk = jnp.sum(nblk)
    act = (bi[:, :1] < totblk).astype(jnp.float32)
    meta_ref[pl.ds(0, _NBP), :] = be.astype(jnp.int32)
    meta_ref[pl.ds(_NBP, _NBP), :] = act.astype(jnp.int32)


def _routing(logits, router_bias):
    return pl.pallas_call(
        _routing_body,
        in_specs=[
            pl.BlockSpec((_N, _E), lambda: (0, 0)),
            pl.BlockSpec((1, _E), lambda: (0, 0)),
        ],
        out_specs=[
            pl.BlockSpec((_K * _N, 1), lambda: (0, 0)),
            pl.BlockSpec((_K * _N, _RW), lambda: (0, 0)),
            pl.BlockSpec((2 * _NBP, 1), lambda: (0, 0)),
        ],
        out_shape=[
            jax.ShapeDtypeStruct((_K * _N, 1), jnp.int32),
            jax.ShapeDtypeStruct((_K * _N, _RW), jnp.float32),
            jax.ShapeDtypeStruct((2 * _NBP, 1), jnp.int32),
        ],
    )(logits, router_bias.reshape(1, _E))


def _dispatch_body(pos_hbm, rec_hbm, recs_hbm, idx_v, rec_v, semr):
    wid = lax.axis_index("s") * 2 + lax.axis_index("c")
    npa = _K * _N // 32
    a0 = wid * npa
    pltpu.sync_copy(pos_hbm.at[pl.ds(a0, npa)], idx_v)
    pltpu.sync_copy(rec_hbm.at[pl.ds(a0, npa)], rec_v)
    pltpu.async_copy(rec_v, recs_hbm.at[idx_v], semr).wait()


def _dispatch_sc(pos_flat, rec):
    npa = _K * _N // 32
    mesh = plsc.VectorSubcoreMesh(core_axis_name="c", subcore_axis_name="s")
    f = functools.partial(
        pl.kernel,
        out_type=jax.ShapeDtypeStruct((_NP, _RW), jnp.float32),
        mesh=mesh,
        scratch_types=[
            pltpu.VMEM((npa,), jnp.int32),
            pltpu.VMEM((npa, _RW), jnp.float32),
            pltpu.SemaphoreType.DMA,
        ],
        compiler_params=pltpu.CompilerParams(use_tc_tiling_on_sc=True),
    )(_dispatch_body)
    return f(pos_flat, rec)


def _ffn_triple(x_blk, wu, wg, wd):
    up = _dot_t(x_blk, wu)
    gate = _dot_t(x_blk, wg)
    hid = _silu(up) * gate
    return lax.dot_general(hid.astype(jnp.bfloat16), wd,
                           (((1,), (0,)), ((), ())),
                           preferred_element_type=jnp.float32)


def _grouped_body(sp_ref, xbf_ref, rec_ref, wu_ref, wg_ref, wd_ref,
                  su_ref, sg_ref, sd_ref, eo_ref, sh_ref):
    b = pl.program_id(0)
    act = sp_ref[_NBP + b]

    @pl.when(b < _NB)
    def _():
        @pl.when(act == 1)
        def _():
            st_blk = rec_ref[:, 0:1]
            ws_blk = rec_ref[:, 1:2]
            toks = lax.broadcasted_iota(
                jnp.int32, (_TBG, _N), 1).astype(jnp.float32)
            p = (toks == st_blk).astype(jnp.bfloat16)
            xg = lax.dot_general(p, xbf_ref[...],
                                 (((1,), (0,)), ((), ())),
                                 preferred_element_type=jnp.float32)
            x_blk = xg.astype(jnp.bfloat16)
            eo = _ffn_triple(x_blk, wu_ref[0], wg_ref[0], wd_ref[0])
            eo_ref[...] = eo * ws_blk

        @pl.when(act == 0)
        def _():
            eo_ref[...] = jnp.zeros_like(eo_ref)

    @pl.when(b >= _NB)
    def _():
        tb = b - _NB
        x_blk = xbf_ref[pl.ds(tb * _TBG, _TBG), :]
        acc = None
        for s in range(su_ref.shape[0]):
            eo = _ffn_triple(x_blk, su_ref[s], sg_ref[s], sd_ref[s])
            acc = eo if acc is None else acc + eo
        sh_ref[...] = acc


def _grouped(meta, xbf, recs, up_r, gate_r, down_r,
             s_up, s_gate, s_down):
    nbt = _NB + _N // _TBG
    grid_spec = pltpu.PrefetchScalarGridSpec(
        num_scalar_prefetch=1,
        grid=(nbt,),
        in_specs=[
            pl.BlockSpec((_N, _D), lambda b, sp: (0, 0)),
            pl.BlockSpec((_TBG, _RW),
                         lambda b, sp: (jnp.minimum(b, _NB - 1), 0)),
            pl.BlockSpec((1, _H, _D), lambda b, sp: (sp[b], 0, 0)),
            pl.BlockSpec((1, _H, _D), lambda b, sp: (sp[b], 0, 0)),
            pl.BlockSpec((1, _H, _D), lambda b, sp: (sp[b], 0, 0)),
            pl.BlockSpec((2, _H, _D), lambda b, sp: (0, 0, 0)),
            pl.BlockSpec((2, _H, _D), lambda b, sp: (0, 0, 0)),
            pl.BlockSpec((2, _H, _D), lambda b, sp: (0, 0, 0)),
        ],
        out_specs=[
            pl.BlockSpec((_TBG, _D),
                         lambda b, sp: (jnp.minimum(b, _NB - 1), 0)),
            pl.BlockSpec((_TBG, _D),
                         lambda b, sp: (jnp.maximum(b - _NB, 0), 0)),
        ],
    )
    return pl.pallas_call(
        _grouped_body,
        grid_spec=grid_spec,
        out_shape=[
            jax.ShapeDtypeStruct((_NP, _D), jnp.float32),
            jax.ShapeDtypeStruct((_N, _D), jnp.float32),
        ],
        compiler_params=pltpu.CompilerParams(
            dimension_semantics=("arbitrary",)),
    )(meta.reshape(-1), xbf, recs, up_r, gate_r, down_r,
      s_up, s_gate, s_down)


_CSZ = 8


def _combine_body(eo_hbm, sh_hbm, pos_hbm, out_hbm,
                  idx0a, idx1a, g0a, g1a, sha,
                  idx0b, idx1b, g0b, g1b, shb,
                  o_v, sem0a, sem1a, semsa, sem0b, sem1b, semsb):
    wid = lax.axis_index("s") * 2 + lax.axis_index("c")
    tpw = _N // 32
    nch = tpw // _CSZ
    bufs = [(idx0a, idx1a, g0a, g1a, sha, sem0a, sem1a, semsa),
            (idx0b, idx1b, g0b, g1b, shb, sem0b, sem1b, semsb)]

    def start(ch):
        i0, i1, g0, g1, sh, s0, s1, ss = bufs[ch % 2]
        t0 = wid * tpw + ch * _CSZ
        pltpu.sync_copy(pos_hbm.at[pl.ds(t0, _CSZ)], i0)
        pltpu.sync_copy(pos_hbm.at[pl.ds(_N + t0, _CSZ)], i1)
        c0 = pltpu.async_copy(eo_hbm.at[i0], g0, s0)
        c1 = pltpu.async_copy(eo_hbm.at[i1], g1, s1)
        cs = pltpu.async_copy(sh_hbm.at[pl.ds(t0, _CSZ)], sh, ss)
        return c0, c1, cs

    pend = start(0)
    for ch in range(nch):
        nxt = start(ch + 1) if ch + 1 < nch else None
        for c in pend:
            c.wait()
        _, _, g0, g1, sh, _, _, _ = bufs[ch % 2]

        def body(c, carry):
            for r in range(_CSZ):
                sl = pl.ds(c * 16, 16)
                o_v[r, sl] = g0[r, sl] + g1[r, sl] + sh[r, sl]
            return carry

        lax.fori_loop(0, _D // 16, body, 0)
        t0 = wid * tpw + ch * _CSZ
        pltpu.sync_copy(o_v, out_hbm.at[pl.ds(t0, _CSZ)])
        pend = nxt


def _combine_sc(eo, shared_out, pos_flat):
    mesh = plsc.VectorSubcoreMesh(core_axis_name="c", subcore_axis_name="s")
    buf = lambda: pltpu.VMEM((_CSZ, _D), jnp.float32)
    idx = lambda: pltpu.VMEM((_CSZ,), jnp.int32)
    f = functools.partial(
        pl.kernel,
        out_type=jax.ShapeDtypeStruct((_N, _D), jnp.float32),
        mesh=mesh,
        scratch_types=[
            idx(), idx(), buf(), buf(), buf(),
            idx(), idx(), buf(), buf(), buf(),
            buf(),
            pltpu.SemaphoreType.DMA, pltpu.SemaphoreType.DMA,
            pltpu.SemaphoreType.DMA, pltpu.SemaphoreType.DMA,
            pltpu.SemaphoreType.DMA, pltpu.SemaphoreType.DMA,
        ],
        compiler_params=pltpu.CompilerParams(use_tc_tiling_on_sc=True),
    )(_combine_body)
    return f(eo, shared_out, pos_flat)


@jax.jit
def kernel(x, Wr, router_bias, W_up, W_gate, W_down, Ws_up, Ws_gate, Ws_down):
    Bn, Tn, Dn = x.shape
    bf = jnp.bfloat16
    flat = x.reshape(_N, _D)
    logits = flat @ Wr.T
    xbf = flat.astype(bf)

    up_r = W_up.astype(bf)
    gate_r = W_gate.astype(bf)
    down_r = W_down.astype(bf).transpose(0, 2, 1)
    s_up = Ws_up[0].astype(bf).reshape(2, _H, _D)
    s_gate = Ws_gate[0].astype(bf).reshape(2, _H, _D)
    s_down = Ws_down[0].astype(bf).T.reshape(2, _H, _D)

    pos, rec, meta = _routing(logits, router_bias)
    pos_flat = pos.reshape(-1)

    recs = _dispatch_sc(pos_flat, rec)
    eo, sh = _grouped(meta, xbf, recs,
                      up_r, gate_r, down_r, s_up, s_gate, s_down)
    out = _combine_sc(eo, sh, pos_flat)
    return out.reshape(Bn, Tn, Dn)
